# Initial kernel scaffold; baseline (speedup 1.0000x reference)
import jax
import jax.numpy as jnp
from jax.experimental import pallas as pl


def kernel(x, node_embedding):
    raise NotImplementedError("write your pallas kernel here")


# speedup vs baseline: 1.2058x; 1.2058x over previous
import functools

import jax
import jax.numpy as jnp
from jax import lax
from jax.experimental import pallas as pl
from jax.experimental.pallas import tpu as pltpu
from jax.experimental.pallas import tpu_sc as plsc

_B = 4096
_D = 64
_NE = 3
_NC, _NS = 2, 16
_NW = _NC * _NS
_BPW = _B // _NW
_L = 16


def _body(xt, table, out, i0, i1, i2, r0, r1, r2, q, ov, sem):
    wid = lax.axis_index("s") * _NC + lax.axis_index("c")
    base = wid * _BPW
    pltpu.sync_copy(xt.at[pl.ds(base, _BPW)], i0)
    pltpu.sync_copy(xt.at[pl.ds(_B + base, _BPW)], i1)
    pltpu.sync_copy(xt.at[pl.ds(2 * _B + base, _BPW)], i2)
    c0 = pltpu.async_copy(table.at[i0], r0, sem)
    c1 = pltpu.async_copy(table.at[i1], r1, sem)
    c2 = pltpu.async_copy(table.at[i2], r2, sem)
    c0.wait()
    c1.wait()
    c2.wait()

    rows = lax.iota(jnp.int32, _L)

    def elem(b, carry):
        acc = r0[b, pl.ds(0, _L)] * r1[b, pl.ds(0, _L)] * r2[b, pl.ds(0, _L)]
        for c in range(1, _D // _L):
            s = pl.ds(c * _L, _L)
            acc = acc + r0[b, s] * r1[b, s] * r2[b, s]
        plsc.store_scatter(q, [rows, jnp.full((_L,), b, jnp.int32)], acc)
        return carry

    lax.fori_loop(0, _BPW, elem, 0)

    for g in range(_BPW // _L):
        s = pl.ds(g * _L, _L)
        tot = q[0, s]
        for r in range(1, _L):
            tot = tot + q[r, s]
        ov[s] = 1.0 / (1.0 + jnp.exp(-tot))
    pltpu.sync_copy(ov, out.at[pl.ds(base, _BPW)])


@functools.partial(jax.jit, static_argnames=())
def _run(xt, table):
    mesh = plsc.VectorSubcoreMesh(
        core_axis_name="c", subcore_axis_name="s",
        num_cores=_NC, num_subcores=_NS,
    )
    return pl.kernel(
        _body,
        out_type=jax.ShapeDtypeStruct((_B,), jnp.float32),
        mesh=mesh,
        compiler_params=pltpu.CompilerParams(
            needs_layout_passes=False, use_tc_tiling_on_sc=False),
        scratch_types=[
            pltpu.VMEM((_BPW,), jnp.int32),
            pltpu.VMEM((_BPW,), jnp.int32),
            pltpu.VMEM((_BPW,), jnp.int32),
            pltpu.VMEM((_BPW, _D), jnp.float32),
            pltpu.VMEM((_BPW, _D), jnp.float32),
            pltpu.VMEM((_BPW, _D), jnp.float32),
            pltpu.VMEM((_L, _BPW), jnp.float32),
            pltpu.VMEM((_BPW,), jnp.float32),
            pltpu.SemaphoreType.DMA,
        ],
    )(xt, table)


def kernel(x, node_embedding):
    xt = x.astype(jnp.int32).T.reshape(-1)
    return _run(xt, node_embedding)


# speedup vs baseline: 1.2145x; 1.0073x over previous
import functools

import jax
import jax.numpy as jnp
from jax import lax
from jax.experimental import pallas as pl
from jax.experimental.pallas import tpu as pltpu
from jax.experimental.pallas import tpu_sc as plsc

_B = 4096
_D = 64
_NE = 3
_NC, _NS = 2, 16
_NW = _NC * _NS
_BPW = _B // _NW
_L = 16
_IPW = _BPW * _NE


def _body(xf, table, out, ia, rows_v, q, ov, sem):
    wid = lax.axis_index("s") * _NC + lax.axis_index("c")
    base = wid * _BPW
    for c in range(_NE):
        pltpu.sync_copy(xf.at[pl.ds(wid * _IPW + c * _BPW, _BPW)], ia.at[c])
    dmas = [
        pltpu.async_copy(table.at[ia.at[c]],
                         rows_v.at[pl.ds(c * _BPW, _BPW)], sem)
        for c in range(_NE)
    ]
    for d in dmas:
        d.wait()

    lanes = lax.iota(jnp.int32, _L)

    def elem(b, carry):
        p = _NE * b
        acc = rows_v[p, pl.ds(0, _L)] * rows_v[p + 1, pl.ds(0, _L)] \
            * rows_v[p + 2, pl.ds(0, _L)]
        for k in range(1, _D // _L):
            s = pl.ds(k * _L, _L)
            acc = acc + rows_v[p, s] * rows_v[p + 1, s] * rows_v[p + 2, s]
        plsc.store_scatter(q, [lanes, jnp.full((_L,), b, jnp.int32)], acc)
        return carry

    lax.fori_loop(0, _BPW, elem, 0)

    for g in range(_BPW // _L):
        s = pl.ds(g * _L, _L)
        tot = q[0, s]
        for r in range(1, _L):
            tot = tot + q[r, s]
        ov[s] = 1.0 / (1.0 + jnp.exp(-tot))
    pltpu.sync_copy(ov, out.at[pl.ds(base, _BPW)])


@functools.partial(jax.jit, static_argnames=())
def _run(xf, table):
    mesh = plsc.VectorSubcoreMesh(
        core_axis_name="c", subcore_axis_name="s",
        num_cores=_NC, num_subcores=_NS,
    )
    return pl.kernel(
        _body,
        out_type=jax.ShapeDtypeStruct((_B,), jnp.float32),
        mesh=mesh,
        compiler_params=pltpu.CompilerParams(
            needs_layout_passes=False, use_tc_tiling_on_sc=False),
        scratch_types=[
            pltpu.VMEM((_NE, _BPW), jnp.int32),
            pltpu.VMEM((_IPW, _D), jnp.float32),
            pltpu.VMEM((_L, _BPW), jnp.float32),
            pltpu.VMEM((_BPW,), jnp.float32),
            pltpu.SemaphoreType.DMA,
        ],
    )(xf, table)


def kernel(x, node_embedding):
    xf = x.astype(jnp.int32).reshape(-1)
    return _run(xf, node_embedding)


# speedup vs baseline: 2.1363x; 1.7590x over previous
import functools

import jax
import jax.numpy as jnp
from jax import lax
from jax.experimental import pallas as pl
from jax.experimental.pallas import tpu as pltpu
from jax.experimental.pallas import tpu_sc as plsc

_B = 4096
_D = 64
_NE = 3
_NC, _NS = 2, 16
_NW = _NC * _NS
_BPW = _B // _NW
_L = 16
_IPW = _BPW * _NE
_RG = _IPW // 8


def _body(xf, tbl3, out, iv, ism, rows_v, q, ov, sem):
    wid = lax.axis_index("s") * _NC + lax.axis_index("c")
    base = wid * _BPW
    pltpu.sync_copy(xf.at[pl.ds(wid * _IPW, _IPW)], iv)

    def fire(g, carry):
        vec = iv[pl.ds(g * _L, _L)]
        tvec = lax.shift_right_logical(vec, 3)
        rvec = lax.bitwise_and(vec, 7)
        for e in range(_L):
            pltpu.async_copy(
                tbl3.at[tvec[e], rvec[e]],
                rows_v.at[2 * g + e // 8, e % 8],
                sem)
        return carry

    lax.fori_loop(0, _IPW // _L, fire, 0)
    pltpu.make_async_copy(tbl3.at[pl.ds(0, _RG)], rows_v, sem).wait()

    lanes = lax.iota(jnp.int32, _L)

    def elem(b, carry):
        p = _NE * b
        acc = None
        for k in range(_D // _L):
            s = pl.ds(k * _L, _L)
            t = rows_v[lax.shift_right_logical(p, 3),
                       lax.bitwise_and(p, 7), s] \
                * rows_v[lax.shift_right_logical(p + 1, 3),
                         lax.bitwise_and(p + 1, 7), s] \
                * rows_v[lax.shift_right_logical(p + 2, 3),
                         lax.bitwise_and(p + 2, 7), s]
            acc = t if acc is None else acc + t
        plsc.store_scatter(q, [lanes, jnp.full((_L,), b, jnp.int32)], acc)
        return carry

    lax.fori_loop(0, _BPW, elem, 0)

    for g in range(_BPW // _L):
        s = pl.ds(g * _L, _L)
        tot = q[0, s]
        for r in range(1, _L):
            tot = tot + q[r, s]
        ov[s] = 1.0 / (1.0 + jnp.exp(-tot))
    pltpu.sync_copy(ov, out.at[pl.ds(base, _BPW)])


@functools.partial(jax.jit, static_argnames=())
def _run(xf, tbl3):
    mesh = plsc.VectorSubcoreMesh(
        core_axis_name="c", subcore_axis_name="s",
        num_cores=_NC, num_subcores=_NS,
    )
    return pl.kernel(
        _body,
        out_type=jax.ShapeDtypeStruct((_B,), jnp.float32),
        mesh=mesh,
        compiler_params=pltpu.CompilerParams(
            needs_layout_passes=False, use_tc_tiling_on_sc=True),
        scratch_types=[
            pltpu.VMEM((_IPW,), jnp.int32),
            pltpu.SMEM((_IPW,), jnp.int32),
            pltpu.VMEM((_RG, 8, _D), jnp.float32),
            pltpu.VMEM((_L, _BPW), jnp.float32),
            pltpu.VMEM((_BPW,), jnp.float32),
            pltpu.SemaphoreType.DMA,
        ],
    )(xf, tbl3)


def kernel(x, node_embedding):
    xf = x.astype(jnp.int32).reshape(-1)
    v = node_embedding.shape[0]
    tbl3 = node_embedding.reshape(v // 8, 8, _D)
    return _run(xf, tbl3)
